# baseline (speedup 1.0000x reference)
import functools

import jax
import jax.numpy as jnp
from jax import lax
from jax.experimental import pallas as pl
from jax.experimental.pallas import tpu as pltpu
from jax.experimental.pallas import tpu_sc as plsc

_MAX_LEN = 2048
_D_MODEL = 1024
_BATCH = 4

_NC = 2
_NS = 16
_NW = _NC * _NS
_LANES = 16

_ROWS_PER_W = _MAX_LEN // _NW
_CHUNK_ROWS = 16
_NCHUNK = _ROWS_PER_W // _CHUNK_ROWS
_CW = _CHUNK_ROWS * _D_MODEL
_NSTEP = _BATCH * _NCHUNK
_NBUF = 5
_LOOKAHEAD = 3


@functools.partial(
    pl.kernel,
    mesh=plsc.VectorSubcoreMesh(core_axis_name="c", subcore_axis_name="s"),
    out_type=jax.ShapeDtypeStruct((_BATCH, _MAX_LEN, _D_MODEL), jnp.float32),
    scratch_types=(
        [pltpu.VMEM((_CHUNK_ROWS, _D_MODEL), jnp.float32)] * 2
        + [pltpu.VMEM((_CHUNK_ROWS, _D_MODEL), jnp.float32)] * _NBUF
        + [pltpu.SemaphoreType.DMA] * (2 + 2 * _NBUF)
    ),
)
def _posemb_add(x_hbm, t_hbm, out_hbm, *scratch):
    tbufs = scratch[:2]
    xbufs = scratch[2:2 + _NBUF]
    tsems = scratch[2 + _NBUF:4 + _NBUF]
    insems = scratch[4 + _NBUF:4 + 2 * _NBUF]
    outsems = scratch[4 + 2 * _NBUF:]

    wid = lax.axis_index("s") * _NC + lax.axis_index("c")
    row0 = wid * _ROWS_PER_W

    def rows(s):
        q, b = divmod(s, _BATCH)
        return b, pl.ds(row0 + q * _CHUNK_ROWS, _CHUNK_ROWS)

    def start_in(s):
        b, sl = rows(s)
        return pltpu.async_copy(
            x_hbm.at[b, sl, :], xbufs[s % _NBUF], insems[s % _NBUF])

    def start_t(q):
        return pltpu.async_copy(
            t_hbm.at[pl.ds(row0 + q * _CHUNK_ROWS, _CHUNK_ROWS), :],
            tbufs[q % 2], tsems[q % 2])

    th = {0: start_t(0), 1: start_t(1)}
    inh = {s: start_in(s) for s in range(_LOOKAHEAD)}
    outh = {}
    for s in range(_NSTEP):
        bi = s % _NBUF
        q, b = divmod(s, _BATCH)
        if b == 0:
            th[q].wait()
        tb = tbufs[q % 2]
        inh[s].wait()
        xb = xbufs[bi]

        @plsc.parallel_loop(0, _CW, step=_LANES, unroll=2)
        def _(j):
            r = jax.lax.shift_right_logical(j, 10)
            c = pl.multiple_of(jax.lax.bitwise_and(j, _D_MODEL - 1), _LANES)
            plsc.addupdate(xb.at[r, pl.ds(c, _LANES)], tb[r, pl.ds(c, _LANES)])

        bb, sl = rows(s)
        outh[s] = pltpu.async_copy(xb, out_hbm.at[bb, sl, :], outsems[bi])
        if b == _BATCH - 1 and q + 2 < _NCHUNK:
            th[q + 2] = start_t(q + 2)
        if s + _LOOKAHEAD < _NSTEP:
            if s + _LOOKAHEAD >= _NBUF:
                outh[s + _LOOKAHEAD - _NBUF].wait()
            inh[s + _LOOKAHEAD] = start_in(s + _LOOKAHEAD)
    for s in range(_NSTEP - _NBUF, _NSTEP):
        outh[s].wait()


def kernel(x, table):
    return _posemb_add(x, table)


# speedup vs baseline: 1.1945x; 1.1945x over previous
import functools

import jax
import jax.numpy as jnp
from jax import lax
from jax.experimental import pallas as pl
from jax.experimental.pallas import tpu as pltpu
from jax.experimental.pallas import tpu_sc as plsc

_MAX_LEN = 2048
_D_MODEL = 1024
_BATCH = 4

_NC = 2
_NS = 16
_NW = _NC * _NS
_LANES = 16

_ROWS_PER_W = _MAX_LEN // _NW
_CHUNK_ROWS = 16
_NCHUNK = _ROWS_PER_W // _CHUNK_ROWS
_CW = _CHUNK_ROWS * _D_MODEL
_NSTEP = _BATCH * _NCHUNK
_NBUF = 5
_LOOKAHEAD = 3


@functools.partial(
    pl.kernel,
    mesh=plsc.VectorSubcoreMesh(core_axis_name="c", subcore_axis_name="s"),
    out_type=jax.ShapeDtypeStruct((_BATCH, _MAX_LEN, _D_MODEL), jnp.float32),
    scratch_types=[
        pltpu.VMEM((2, _CHUNK_ROWS, _D_MODEL), jnp.float32),
        pltpu.VMEM((_NBUF, _CHUNK_ROWS, _D_MODEL), jnp.float32),
        pltpu.SemaphoreType.DMA,
        pltpu.SemaphoreType.DMA,
        pltpu.SemaphoreType.DMA,
    ],
)
def _posemb_add(x_hbm, t_hbm, out_hbm, tball, xball, tsem, insem, outsem):
    wid = lax.axis_index("s") * _NC + lax.axis_index("c")
    row0 = wid * _ROWS_PER_W

    def x_slice(s):
        q = jax.lax.shift_right_logical(s, 2)
        b = jax.lax.bitwise_and(s, _BATCH - 1)
        return b, pl.ds(row0 + q * _CHUNK_ROWS, _CHUNK_ROWS)

    def start_in(s, bi):
        b, sl = x_slice(s)
        return pltpu.async_copy(x_hbm.at[b, sl, :], xball.at[bi], insem)

    def start_t(q, ti):
        return pltpu.async_copy(
            t_hbm.at[pl.ds(row0 + q * _CHUNK_ROWS, _CHUNK_ROWS), :],
            tball.at[ti], tsem)

    start_t(jnp.int32(0), jnp.int32(0))
    start_t(jnp.int32(1), jnp.int32(1))
    for s in range(_LOOKAHEAD):
        start_in(jnp.int32(s), jnp.int32(s))

    def wait_in(bi):
        pltpu.make_async_copy(x_hbm.at[0, pl.ds(0, _CHUNK_ROWS), :],
                              xball.at[bi], insem).wait()

    def wait_out(bi):
        pltpu.make_async_copy(xball.at[bi],
                              out_hbm.at[0, pl.ds(0, _CHUNK_ROWS), :],
                              outsem).wait()

    def wait_t(ti):
        pltpu.make_async_copy(t_hbm.at[pl.ds(0, _CHUNK_ROWS), :],
                              tball.at[ti], tsem).wait()

    def body(s, bi):
        q = jax.lax.shift_right_logical(s, 2)
        b = jax.lax.bitwise_and(s, _BATCH - 1)
        ti = jax.lax.bitwise_and(q, 1)

        @pl.when(b == 0)
        def _():
            wait_t(ti)

        wait_in(bi)

        @plsc.parallel_loop(0, _CW, step=_LANES, unroll=4)
        def _(j):
            r = jax.lax.shift_right_logical(j, 10)
            c = pl.multiple_of(jax.lax.bitwise_and(j, _D_MODEL - 1), _LANES)
            plsc.addupdate(xball.at[bi, r, pl.ds(c, _LANES)],
                           tball[ti, r, pl.ds(c, _LANES)])

        bb, sl = x_slice(s)
        pltpu.async_copy(xball.at[bi], out_hbm.at[bb, sl, :], outsem)

        @pl.when(jnp.logical_and(b == _BATCH - 1, q + 2 < _NCHUNK))
        def _():
            start_t(q + 2, jax.lax.bitwise_and(q, 1))

        @pl.when(s + _LOOKAHEAD < _NSTEP)
        def _():
            pf = s + _LOOKAHEAD
            pfbi = jax.lax.rem(pf, _NBUF)

            @pl.when(pf >= _NBUF)
            def _():
                wait_out(pfbi)

            b2, sl2 = x_slice(pf)
            pltpu.async_copy(x_hbm.at[b2, sl2, :], xball.at[pfbi], insem)

        return jax.lax.select(bi == _NBUF - 1, jnp.int32(0), bi + 1)

    lax.fori_loop(0, _NSTEP, body, jnp.int32(0))
    for _ in range(_NBUF):
        wait_out(jnp.int32(0))


def kernel(x, table):
    return _posemb_add(x, table)


# speedup vs baseline: 1.1960x; 1.0012x over previous
import functools

import jax
import jax.numpy as jnp
from jax import lax
from jax.experimental import pallas as pl
from jax.experimental.pallas import tpu as pltpu
from jax.experimental.pallas import tpu_sc as plsc

_MAX_LEN = 2048
_D_MODEL = 1024
_BATCH = 4

_NC = 2
_NS = 16
_NW = _NC * _NS
_LANES = 16

_ROWS_PER_W = _MAX_LEN // _NW
_CHUNK_ROWS = 16
_NCHUNK = _ROWS_PER_W // _CHUNK_ROWS
_CW = _CHUNK_ROWS * _D_MODEL
_NSTEP = _BATCH * _NCHUNK
_NBUF = 5
_LOOKAHEAD = 3


@functools.partial(
    pl.kernel,
    mesh=plsc.VectorSubcoreMesh(core_axis_name="c", subcore_axis_name="s"),
    out_type=jax.ShapeDtypeStruct((_BATCH, _MAX_LEN, _D_MODEL), jnp.float32),
    scratch_types=[
        pltpu.VMEM((2, _CHUNK_ROWS, _D_MODEL), jnp.float32),
        pltpu.VMEM((_NBUF, _CHUNK_ROWS, _D_MODEL), jnp.float32),
        pltpu.SemaphoreType.DMA,
        pltpu.SemaphoreType.DMA,
        pltpu.SemaphoreType.DMA,
    ],
)
def _posemb_add(x_hbm, t_hbm, out_hbm, tball, xball, tsem, insem, outsem):
    wid = lax.axis_index("c") * _NS + lax.axis_index("s")
    row0 = wid * _ROWS_PER_W

    def x_slice(s):
        q = jax.lax.shift_right_logical(s, 2)
        b = jax.lax.bitwise_and(s, _BATCH - 1)
        return b, pl.ds(row0 + q * _CHUNK_ROWS, _CHUNK_ROWS)

    def start_in(s, bi):
        b, sl = x_slice(s)
        return pltpu.async_copy(x_hbm.at[b, sl, :], xball.at[bi], insem)

    def start_t(q, ti):
        return pltpu.async_copy(
            t_hbm.at[pl.ds(row0 + q * _CHUNK_ROWS, _CHUNK_ROWS), :],
            tball.at[ti], tsem)

    start_t(jnp.int32(0), jnp.int32(0))
    start_t(jnp.int32(1), jnp.int32(1))
    for s in range(_LOOKAHEAD):
        start_in(jnp.int32(s), jnp.int32(s))

    def wait_in(bi):
        pltpu.make_async_copy(x_hbm.at[0, pl.ds(0, _CHUNK_ROWS), :],
                              xball.at[bi], insem).wait()

    def wait_out(bi):
        pltpu.make_async_copy(xball.at[bi],
                              out_hbm.at[0, pl.ds(0, _CHUNK_ROWS), :],
                              outsem).wait()

    def wait_t(ti):
        pltpu.make_async_copy(t_hbm.at[pl.ds(0, _CHUNK_ROWS), :],
                              tball.at[ti], tsem).wait()

    def body(s, bi):
        q = jax.lax.shift_right_logical(s, 2)
        b = jax.lax.bitwise_and(s, _BATCH - 1)
        ti = jax.lax.bitwise_and(q, 1)

        @pl.when(b == 0)
        def _():
            wait_t(ti)

        wait_in(bi)

        @plsc.parallel_loop(0, _CW, step=_LANES, unroll=4)
        def _(j):
            r = jax.lax.shift_right_logical(j, 10)
            c = pl.multiple_of(jax.lax.bitwise_and(j, _D_MODEL - 1), _LANES)
            plsc.addupdate(xball.at[bi, r, pl.ds(c, _LANES)],
                           tball[ti, r, pl.ds(c, _LANES)])

        bb, sl = x_slice(s)
        pltpu.async_copy(xball.at[bi], out_hbm.at[bb, sl, :], outsem)

        @pl.when(jnp.logical_and(b == _BATCH - 1, q + 2 < _NCHUNK))
        def _():
            start_t(q + 2, jax.lax.bitwise_and(q, 1))

        @pl.when(s + _LOOKAHEAD < _NSTEP)
        def _():
            pf = s + _LOOKAHEAD
            pfbi = jax.lax.rem(pf, _NBUF)

            @pl.when(pf >= _NBUF)
            def _():
                wait_out(pfbi)

            b2, sl2 = x_slice(pf)
            pltpu.async_copy(x_hbm.at[b2, sl2, :], xball.at[pfbi], insem)

        return jax.lax.select(bi == _NBUF - 1, jnp.int32(0), bi + 1)

    lax.fori_loop(0, _NSTEP, body, jnp.int32(0))
    for _ in range(_NBUF):
        wait_out(jnp.int32(0))


def kernel(x, table):
    return _posemb_add(x, table)
